# baseline (speedup 1.0000x reference)
import functools

import jax
import jax.numpy as jnp
from jax import lax
from jax.experimental import pallas as pl
from jax.experimental.pallas import tpu as pltpu
from jax.experimental.pallas import tpu_sc as plsc

L = 16
NC, NS = 2, 16
NW = NC * NS
B = 16384
F = 64
N = 1000000
BPW = B // NW
GROUPS = BPW // L

CW = 512
NCHUNK = N // CW
CPW = (NCHUNK + NW - 1) // NW
TAIL0 = NCHUNK * CW
SINK = B

_mesh = plsc.VectorSubcoreMesh(core_axis_name="c", subcore_axis_name="s")

_ROWS_T = jax.ShapeDtypeStruct((B + L, 2 * F), jnp.float32)


@functools.partial(
    pl.kernel,
    out_type=(_ROWS_T, _ROWS_T),
    mesh=_mesh,
    compiler_params=pltpu.CompilerParams(
        needs_layout_passes=False, use_tc_tiling_on_sc=True
    ),
    scratch_types=[
        pltpu.VMEM((B,), jnp.int32),
        pltpu.VMEM((B + L,), jnp.int32),
        pltpu.VMEM((B + L,), jnp.int32),
        pltpu.VMEM((B + L,), jnp.int32),
        pltpu.VMEM((B + L,), jnp.int32),
        pltpu.VMEM((8, 8, CW), jnp.float32),
        pltpu.VMEM((L, 2 * F), jnp.float32),
        pltpu.VMEM((L,), jnp.int32),
        pltpu.VMEM((64, F), jnp.float32),
        pltpu.SemaphoreType.DMA,
        pltpu.SemaphoreType.DMA,
    ],
)
def _extract_kernel(users_hbm, items_hbm, tfu_hbm, tfi_hbm, tailu_hbm,
                    taili_hbm, urows_hbm, irows_hbm,
                    allidx, lrow, lpos, hrow, hpos, buf, rowbuf, pidx,
                    tailbuf, sem, sem2):
    wid = lax.axis_index("s") * NC + lax.axis_index("c")
    c_lo = wid * CPW

    lane = lax.iota(jnp.int32, L)
    fq = [lane + q * L for q in range(F // L)]
    band_q = [f >> 3 for f in fq]
    k_q = [f & 7 for f in fq]

    for (idx_hbm, tf_hbm, tail_hbm, out_hbm) in (
        (users_hbm, tfu_hbm, tailu_hbm, urows_hbm),
        (items_hbm, tfi_hbm, taili_hbm, irows_hbm),
    ):
        pltpu.sync_copy(idx_hbm, allidx)

        def filt(g, cnt):
            v = allidx[pl.ds(g * L, L)]
            c_of = jax.lax.shift_right_logical(v, 9)
            m = (c_of >= c_lo) & (c_of < c_lo + CPW)
            plsc.store_compressed(lrow.at[pl.ds(cnt, L)], v, mask=m)
            plsc.store_compressed(lpos.at[pl.ds(cnt, L)], g * L + lane, mask=m)
            return cnt + plsc.all_reduce_population_count(m)[0]

        n_local = lax.fori_loop(0, B // L, filt, 0)

        def chunk_body(j, carry):
            c = c_lo + j
            valid = c < NCHUNK

            @pl.when(valid)
            def _():
                cps = [
                    pltpu.async_copy(
                        tf_hbm.at[pl.ds(band * 8, 8), pl.ds(c * CW, CW)],
                        buf.at[band, :, :],
                        sem,
                    )
                    for band in range(8)
                ]
                for cp in cps:
                    cp.wait()

            @pl.when(valid)
            def _():
                def rescan(g, cnt):
                    v = lrow[pl.ds(g * L, L)]
                    p = lpos[pl.ds(g * L, L)]
                    m = (jax.lax.shift_right_logical(v, 9) == c) & (
                        g * L + lane < n_local
                    )
                    plsc.store_compressed(hrow.at[pl.ds(cnt, L)], v, mask=m)
                    plsc.store_compressed(hpos.at[pl.ds(cnt, L)], p, mask=m)
                    return cnt + plsc.all_reduce_population_count(m)[0]

                ngr = (n_local + L - 1) // L
                nhit = lax.fori_loop(0, ngr, rescan, 0)

                def hit_group(gi, carry2):
                    kv = gi * L
                    hrv = hrow[pl.ds(kv, L)]
                    hpv = hpos[pl.ds(kv, L)]
                    nvalid = nhit - kv
                    pidx[pl.ds(0, L)] = jnp.where(lane < nvalid, hpv, SINK)
                    for r in range(L):
                        col = hrv[r] - c * CW
                        col = jnp.clip(col, 0, CW - 1)
                        colv = jnp.full((L,), col, jnp.int32)
                        for q in range(F // L):
                            rowbuf[r, pl.ds(q * L, L)] = plsc.load_gather(
                                buf, [band_q[q], k_q[q], colv]
                            )
                    pltpu.async_copy(rowbuf, out_hbm.at[pidx], sem2).wait()
                    return carry2

                nhg = (nhit + L - 1) // L
                lax.fori_loop(0, nhg, hit_group, 0)

            return carry

        lax.fori_loop(0, CPW, chunk_body, 0)

        @pl.when((c_lo <= NCHUNK) & (NCHUNK < c_lo + CPW))
        def _():
            pltpu.sync_copy(tail_hbm, tailbuf)

            def rescan_t(g, cnt):
                v = lrow[pl.ds(g * L, L)]
                p = lpos[pl.ds(g * L, L)]
                m = (jax.lax.shift_right_logical(v, 9) == NCHUNK) & (
                    g * L + lane < n_local
                )
                plsc.store_compressed(hrow.at[pl.ds(cnt, L)], v, mask=m)
                plsc.store_compressed(hpos.at[pl.ds(cnt, L)], p, mask=m)
                return cnt + plsc.all_reduce_population_count(m)[0]

            ngr = (n_local + L - 1) // L
            nhit = lax.fori_loop(0, ngr, rescan_t, 0)

            def hit_group_t(gi, carry2):
                kv = gi * L
                hrv = hrow[pl.ds(kv, L)]
                hpv = hpos[pl.ds(kv, L)]
                nvalid = nhit - kv
                pidx[pl.ds(0, L)] = jnp.where(lane < nvalid, hpv, SINK)
                for r in range(L):
                    row = jnp.clip(hrv[r] - TAIL0, 0, 63)
                    for q in range(F // L):
                        rowbuf[r, pl.ds(q * L, L)] = tailbuf[row, pl.ds(q * L, L)]
                pltpu.async_copy(rowbuf, out_hbm.at[pidx], sem2).wait()
                return carry2

            nhg = (nhit + L - 1) // L
            lax.fori_loop(0, nhg, hit_group_t, 0)


@functools.partial(
    pl.kernel,
    out_type=jax.ShapeDtypeStruct((B,), jnp.float32),
    mesh=_mesh,
    compiler_params=pltpu.CompilerParams(
        needs_layout_passes=False, use_tc_tiling_on_sc=True
    ),
    scratch_types=[
        pltpu.VMEM((BPW // 2, 2 * F), jnp.float32),
        pltpu.VMEM((BPW // 2, 2 * F), jnp.float32),
        pltpu.VMEM((BPW,), jnp.float32),
        pltpu.VMEM((BPW,), jnp.float32),
        pltpu.VMEM((BPW,), jnp.float32),
        pltpu.VMEM((L * (L + 1),), jnp.float32),
        pltpu.SemaphoreType.DMA,
    ],
)
def _dot_kernel(urows_hbm, irows_hbm, ubg_hbm, ibg_hbm,
                out_hbm, urows, irows, ubv, ibv, outv, tbuf, sem):
    wid = lax.axis_index("s") * NC + lax.axis_index("c")
    base = wid * BPW
    HB = BPW // 2

    pltpu.sync_copy(ubg_hbm.at[pl.ds(base, BPW)], ubv)
    pltpu.sync_copy(ibg_hbm.at[pl.ds(base, BPW)], ibv)

    rowi = lax.iota(jnp.int32, L)

    for half in range(2):
        hbase = base + half * HB
        c1 = pltpu.async_copy(urows_hbm.at[pl.ds(hbase, HB), :], urows, sem)
        c2 = pltpu.async_copy(irows_hbm.at[pl.ds(hbase, HB), :], irows, sem)
        c1.wait()
        c2.wait()

        def group_body(g, carry):
            gb = g * L
            ob = half * HB + gb
            for r in range(L):
                b = gb + r
                acc = urows[b, pl.ds(0, L)] * irows[b, pl.ds(0, L)]
                for j in range(1, F // L):
                    acc = acc + urows[b, pl.ds(j * L, L)] * irows[b, pl.ds(j * L, L)]
                tbuf[pl.ds(r * (L + 1), L)] = acc
            out16 = ubv[pl.ds(ob, L)] + ibv[pl.ds(ob, L)]
            flat = rowi * (L + 1)
            for j in range(L):
                out16 = out16 + plsc.load_gather(tbuf, [flat + j])
            outv[pl.ds(ob, L)] = out16
            return carry

        lax.fori_loop(0, HB // L, group_body, 0)

    pltpu.sync_copy(outv, out_hbm.at[pl.ds(base, BPW)])


@functools.partial(
    pl.kernel,
    out_type=(
        jax.ShapeDtypeStruct((B,), jnp.float32),
        jax.ShapeDtypeStruct((B,), jnp.float32),
    ),
    mesh=_mesh,
    compiler_params=pltpu.CompilerParams(
        needs_layout_passes=False, use_tc_tiling_on_sc=False
    ),
    scratch_types=[
        pltpu.VMEM((BPW,), jnp.int32),
        pltpu.VMEM((BPW,), jnp.int32),
        pltpu.VMEM((BPW,), jnp.float32),
        pltpu.VMEM((BPW,), jnp.float32),
        pltpu.SemaphoreType.DMA,
    ],
)
def _bias_kernel(users_hbm, items_hbm, ub_hbm, ib_hbm, ubg_hbm, ibg_hbm,
                 uidx, iidx, ubv, ibv, sem):
    wid = lax.axis_index("s") * NC + lax.axis_index("c")
    base = wid * BPW
    pltpu.sync_copy(users_hbm.at[pl.ds(base, BPW)], uidx)
    pltpu.sync_copy(items_hbm.at[pl.ds(base, BPW)], iidx)
    c1 = pltpu.async_copy(ub_hbm.at[uidx], ubv, sem)
    c2 = pltpu.async_copy(ib_hbm.at[iidx], ibv, sem)
    c1.wait()
    c2.wait()
    pltpu.sync_copy(ubv, ubg_hbm.at[pl.ds(base, BPW)])
    pltpu.sync_copy(ibv, ibg_hbm.at[pl.ds(base, BPW)])


@jax.jit
def kernel(users, items, user_factors, item_factors, user_bias, item_bias):
    tfu = user_factors.T
    tfi = item_factors.T
    ub = user_bias.reshape(-1)
    ib = item_bias.reshape(-1)
    tailu = user_factors[TAIL0:, :]
    taili = item_factors[TAIL0:, :]
    urows_all, irows_all = _extract_kernel(users, items, tfu, tfi, tailu, taili)
    ubg, ibg = _bias_kernel(users, items, ub, ib)
    return _dot_kernel(urows_all, irows_all, ubg, ibg)


# speedup vs baseline: 1.0023x; 1.0023x over previous
import functools

import jax
import jax.numpy as jnp
from jax import lax
from jax.experimental import pallas as pl
from jax.experimental.pallas import tpu as pltpu
from jax.experimental.pallas import tpu_sc as plsc

L = 16
NC, NS = 2, 16
NW = NC * NS
B = 16384
F = 64
N = 1000000
BPW = B // NW
GROUPS = BPW // L

CW = 512
NCHUNK = N // CW
CPW = (NCHUNK + NW - 1) // NW
TAIL0 = NCHUNK * CW
SINK = B

_mesh = plsc.VectorSubcoreMesh(core_axis_name="c", subcore_axis_name="s")

_ROWS_T = jax.ShapeDtypeStruct((B + L, 2 * F), jnp.float32)


@functools.partial(
    pl.kernel,
    out_type=(_ROWS_T, _ROWS_T),
    mesh=_mesh,
    compiler_params=pltpu.CompilerParams(
        needs_layout_passes=False, use_tc_tiling_on_sc=True
    ),
    scratch_types=[
        pltpu.VMEM((B,), jnp.int32),
        pltpu.VMEM((B + L,), jnp.int32),
        pltpu.VMEM((B + L,), jnp.int32),
        pltpu.VMEM((3 * L,), jnp.int32),
        pltpu.VMEM((3 * L,), jnp.int32),
        pltpu.VMEM((F, CW), jnp.float32),
        pltpu.VMEM((F, CW), jnp.float32),
        pltpu.VMEM((L, 2 * F), jnp.float32),
        pltpu.VMEM((L,), jnp.int32),
        pltpu.VMEM((64, F), jnp.float32),
        pltpu.SemaphoreType.DMA,
        pltpu.SemaphoreType.DMA,
        pltpu.SemaphoreType.DMA,
    ],
)
def _extract_kernel(users_hbm, items_hbm, tfu_hbm, tfi_hbm, tailu_hbm,
                    taili_hbm, urows_hbm, irows_hbm,
                    allidx, lrow, lpos, prow, ppos, bufA, bufB, rowbuf, pidx,
                    tailbuf, semA, semB, sem2):
    wid = lax.axis_index("s") * NC + lax.axis_index("c")
    c_lo = wid * CPW

    lane = lax.iota(jnp.int32, L)
    fq = [lane + q * L for q in range(F // L)]

    for (idx_hbm, tf_hbm, tail_hbm, out_hbm) in (
        (users_hbm, tfu_hbm, tailu_hbm, urows_hbm),
        (items_hbm, tfi_hbm, taili_hbm, irows_hbm),
    ):
        pltpu.sync_copy(idx_hbm, allidx)

        def filt(g, cnt):
            v = allidx[pl.ds(g * L, L)]
            c_of = jax.lax.shift_right_logical(v, 9)
            m = (c_of >= c_lo) & (c_of < c_lo + CPW)
            plsc.store_compressed(lrow.at[pl.ds(cnt, L)], v, mask=m)
            plsc.store_compressed(lpos.at[pl.ds(cnt, L)], g * L + lane, mask=m)
            return cnt + plsc.all_reduce_population_count(m)[0]

        n_local = lax.fori_loop(0, B // L, filt, 0)
        ngr = (n_local + L - 1) // L

        def chunk_slice(c):
            return tf_hbm.at[:, pl.ds(c * CW, CW)]

        def emit_hits(buf, c, nvalid):
            hrv = prow[pl.ds(0, L)]
            hpv = ppos[pl.ds(0, L)]
            pidx[pl.ds(0, L)] = jnp.where(lane < nvalid, hpv, SINK)
            for r in range(L):
                col = jnp.clip(hrv[r] - c * CW, 0, CW - 1)
                colv = jnp.full((L,), col, jnp.int32)
                for q in range(F // L):
                    rowbuf[r, pl.ds(q * L, L)] = plsc.load_gather(
                        buf, [fq[q], colv]
                    )
            pltpu.async_copy(rowbuf, out_hbm.at[pidx], sem2).wait()

        def process(buf, c):
            def rescan(g, pcnt):
                v = lrow[pl.ds(g * L, L)]
                p = lpos[pl.ds(g * L, L)]
                m = (jax.lax.shift_right_logical(v, 9) == c) & (
                    g * L + lane < n_local
                )
                plsc.store_compressed(prow.at[pl.ds(pcnt, L)], v, mask=m)
                plsc.store_compressed(ppos.at[pl.ds(pcnt, L)], p, mask=m)
                pcnt = pcnt + plsc.all_reduce_population_count(m)[0]

                def flush(pc):
                    emit_hits(buf, c, L)
                    prow[pl.ds(0, L)] = prow[pl.ds(L, L)]
                    ppos[pl.ds(0, L)] = ppos[pl.ds(L, L)]
                    return pc - L

                return lax.cond(pcnt >= L, flush, lambda pc: pc, pcnt)

            pcnt = lax.fori_loop(0, ngr, rescan, 0)

            @pl.when(pcnt > 0)
            def _():
                emit_hits(buf, c, pcnt)

        @pl.when(c_lo < NCHUNK)
        def _():
            pltpu.async_copy(chunk_slice(c_lo), bufA, semA)

        def pair_body(m2, carry):
            c0 = c_lo + 2 * m2
            c1 = c0 + 1

            @pl.when(c0 < NCHUNK)
            def _():
                pltpu.make_async_copy(chunk_slice(c0), bufA, semA).wait()

                @pl.when(c1 < NCHUNK)
                def _():
                    pltpu.async_copy(chunk_slice(c1), bufB, semB)

                process(bufA, c0)

            @pl.when(c1 < NCHUNK)
            def _():
                pltpu.make_async_copy(chunk_slice(c1), bufB, semB).wait()

                @pl.when((c1 + 1 < NCHUNK) & (2 * m2 + 2 < CPW))
                def _():
                    pltpu.async_copy(chunk_slice(c1 + 1), bufA, semA)

                process(bufB, c1)

            return carry

        lax.fori_loop(0, CPW // 2, pair_body, 0)

        @pl.when((c_lo <= NCHUNK) & (NCHUNK < c_lo + CPW))
        def _():
            pltpu.sync_copy(tail_hbm, tailbuf)

            def emit_hits_t(nvalid):
                hrv = prow[pl.ds(0, L)]
                hpv = ppos[pl.ds(0, L)]
                pidx[pl.ds(0, L)] = jnp.where(lane < nvalid, hpv, SINK)
                for r in range(L):
                    row = jnp.clip(hrv[r] - TAIL0, 0, 63)
                    for q in range(F // L):
                        rowbuf[r, pl.ds(q * L, L)] = tailbuf[row, pl.ds(q * L, L)]
                pltpu.async_copy(rowbuf, out_hbm.at[pidx], sem2).wait()

            def rescan_t(g, pcnt):
                v = lrow[pl.ds(g * L, L)]
                p = lpos[pl.ds(g * L, L)]
                m = (jax.lax.shift_right_logical(v, 9) == NCHUNK) & (
                    g * L + lane < n_local
                )
                plsc.store_compressed(prow.at[pl.ds(pcnt, L)], v, mask=m)
                plsc.store_compressed(ppos.at[pl.ds(pcnt, L)], p, mask=m)
                pcnt = pcnt + plsc.all_reduce_population_count(m)[0]

                def flush(pc):
                    emit_hits_t(L)
                    prow[pl.ds(0, L)] = prow[pl.ds(L, L)]
                    ppos[pl.ds(0, L)] = ppos[pl.ds(L, L)]
                    return pc - L

                return lax.cond(pcnt >= L, flush, lambda pc: pc, pcnt)

            pcnt = lax.fori_loop(0, ngr, rescan_t, 0)

            @pl.when(pcnt > 0)
            def _():
                emit_hits_t(pcnt)


@functools.partial(
    pl.kernel,
    out_type=jax.ShapeDtypeStruct((B,), jnp.float32),
    mesh=_mesh,
    compiler_params=pltpu.CompilerParams(
        needs_layout_passes=False, use_tc_tiling_on_sc=True
    ),
    scratch_types=[
        pltpu.VMEM((BPW // 2, 2 * F), jnp.float32),
        pltpu.VMEM((BPW // 2, 2 * F), jnp.float32),
        pltpu.VMEM((BPW,), jnp.float32),
        pltpu.VMEM((BPW,), jnp.float32),
        pltpu.VMEM((BPW,), jnp.float32),
        pltpu.VMEM((L * (L + 1),), jnp.float32),
        pltpu.SemaphoreType.DMA,
    ],
)
def _dot_kernel(urows_hbm, irows_hbm, ubg_hbm, ibg_hbm,
                out_hbm, urows, irows, ubv, ibv, outv, tbuf, sem):
    wid = lax.axis_index("s") * NC + lax.axis_index("c")
    base = wid * BPW
    HB = BPW // 2

    pltpu.sync_copy(ubg_hbm.at[pl.ds(base, BPW)], ubv)
    pltpu.sync_copy(ibg_hbm.at[pl.ds(base, BPW)], ibv)

    rowi = lax.iota(jnp.int32, L)

    for half in range(2):
        hbase = base + half * HB
        c1 = pltpu.async_copy(urows_hbm.at[pl.ds(hbase, HB), :], urows, sem)
        c2 = pltpu.async_copy(irows_hbm.at[pl.ds(hbase, HB), :], irows, sem)
        c1.wait()
        c2.wait()

        def group_body(g, carry):
            gb = g * L
            ob = half * HB + gb
            for r in range(L):
                b = gb + r
                acc = urows[b, pl.ds(0, L)] * irows[b, pl.ds(0, L)]
                for j in range(1, F // L):
                    acc = acc + urows[b, pl.ds(j * L, L)] * irows[b, pl.ds(j * L, L)]
                tbuf[pl.ds(r * (L + 1), L)] = acc
            out16 = ubv[pl.ds(ob, L)] + ibv[pl.ds(ob, L)]
            flat = rowi * (L + 1)
            for j in range(L):
                out16 = out16 + plsc.load_gather(tbuf, [flat + j])
            outv[pl.ds(ob, L)] = out16
            return carry

        lax.fori_loop(0, HB // L, group_body, 0)

    pltpu.sync_copy(outv, out_hbm.at[pl.ds(base, BPW)])


@functools.partial(
    pl.kernel,
    out_type=(
        jax.ShapeDtypeStruct((B,), jnp.float32),
        jax.ShapeDtypeStruct((B,), jnp.float32),
    ),
    mesh=_mesh,
    compiler_params=pltpu.CompilerParams(
        needs_layout_passes=False, use_tc_tiling_on_sc=False
    ),
    scratch_types=[
        pltpu.VMEM((BPW,), jnp.int32),
        pltpu.VMEM((BPW,), jnp.int32),
        pltpu.VMEM((BPW,), jnp.float32),
        pltpu.VMEM((BPW,), jnp.float32),
        pltpu.SemaphoreType.DMA,
    ],
)
def _bias_kernel(users_hbm, items_hbm, ub_hbm, ib_hbm, ubg_hbm, ibg_hbm,
                 uidx, iidx, ubv, ibv, sem):
    wid = lax.axis_index("s") * NC + lax.axis_index("c")
    base = wid * BPW
    pltpu.sync_copy(users_hbm.at[pl.ds(base, BPW)], uidx)
    pltpu.sync_copy(items_hbm.at[pl.ds(base, BPW)], iidx)
    c1 = pltpu.async_copy(ub_hbm.at[uidx], ubv, sem)
    c2 = pltpu.async_copy(ib_hbm.at[iidx], ibv, sem)
    c1.wait()
    c2.wait()
    pltpu.sync_copy(ubv, ubg_hbm.at[pl.ds(base, BPW)])
    pltpu.sync_copy(ibv, ibg_hbm.at[pl.ds(base, BPW)])


@jax.jit
def kernel(users, items, user_factors, item_factors, user_bias, item_bias):
    tfu = user_factors.T
    tfi = item_factors.T
    ub = user_bias.reshape(-1)
    ib = item_bias.reshape(-1)
    tailu = user_factors[TAIL0:, :]
    taili = item_factors[TAIL0:, :]
    urows_all, irows_all = _extract_kernel(users, items, tfu, tfi, tailu, taili)
    ubg, ibg = _bias_kernel(users, items, ub, ib)
    return _dot_kernel(urows_all, irows_all, ubg, ibg)


# speedup vs baseline: 1.0037x; 1.0013x over previous
import functools

import jax
import jax.numpy as jnp
from jax import lax
from jax.experimental import pallas as pl
from jax.experimental.pallas import tpu as pltpu
from jax.experimental.pallas import tpu_sc as plsc

L = 16
NC, NS = 2, 16
NW = NC * NS
B = 16384
F = 64
N = 1000000
BPW = B // NW
GROUPS = BPW // L

CW = 512
NCHUNK = N // CW
CPW = (NCHUNK + NW - 1) // NW
TAIL0 = NCHUNK * CW
SINK = B

_mesh = plsc.VectorSubcoreMesh(core_axis_name="c", subcore_axis_name="s")

_ROWS_T = jax.ShapeDtypeStruct((B + L, 2 * F), jnp.float32)


@functools.partial(
    pl.kernel,
    out_type=(_ROWS_T, _ROWS_T),
    mesh=_mesh,
    compiler_params=pltpu.CompilerParams(
        needs_layout_passes=False, use_tc_tiling_on_sc=True
    ),
    scratch_types=[
        pltpu.VMEM((B,), jnp.int32),
        pltpu.VMEM((B + L,), jnp.int32),
        pltpu.VMEM((B + L,), jnp.int32),
        pltpu.VMEM((3 * L,), jnp.int32),
        pltpu.VMEM((3 * L,), jnp.int32),
        pltpu.VMEM((2, F, CW), jnp.float32),
        pltpu.VMEM((L, 2 * F), jnp.float32),
        pltpu.VMEM((L,), jnp.int32),
        pltpu.VMEM((64, F), jnp.float32),
        pltpu.SemaphoreType.DMA((2,)),
        pltpu.SemaphoreType.DMA,
    ],
)
def _extract_kernel(users_hbm, items_hbm, tfu_hbm, tfi_hbm, tailu_hbm,
                    taili_hbm, urows_hbm, irows_hbm,
                    allidx, lrow, lpos, prow, ppos, buf2, rowbuf, pidx,
                    tailbuf, sems, sem2):
    wid = lax.axis_index("s") * NC + lax.axis_index("c")
    c_lo = wid * CPW

    lane = lax.iota(jnp.int32, L)
    fq = [lane + q * L for q in range(F // L)]

    for (idx_hbm, tf_hbm, tail_hbm, out_hbm) in (
        (users_hbm, tfu_hbm, tailu_hbm, urows_hbm),
        (items_hbm, tfi_hbm, taili_hbm, irows_hbm),
    ):
        pltpu.sync_copy(idx_hbm, allidx)

        def filt(g, cnt):
            v = allidx[pl.ds(g * L, L)]
            c_of = jax.lax.shift_right_logical(v, 9)
            m = (c_of >= c_lo) & (c_of < c_lo + CPW)
            plsc.store_compressed(lrow.at[pl.ds(cnt, L)], v, mask=m)
            plsc.store_compressed(lpos.at[pl.ds(cnt, L)], g * L + lane, mask=m)
            return cnt + plsc.all_reduce_population_count(m)[0]

        n_local = lax.fori_loop(0, B // L, filt, 0)
        ngr = (n_local + L - 1) // L

        def chunk_slice(c):
            return tf_hbm.at[:, pl.ds(c * CW, CW)]

        def emit_hits(par, c, nvalid):
            hrv = prow[pl.ds(0, L)]
            hpv = ppos[pl.ds(0, L)]
            pidx[pl.ds(0, L)] = jnp.where(lane < nvalid, hpv, SINK)
            parv = jnp.full((L,), par, jnp.int32)
            for r in range(L):
                col = jnp.clip(hrv[r] - c * CW, 0, CW - 1)
                colv = jnp.full((L,), col, jnp.int32)
                for q in range(F // L):
                    rowbuf[r, pl.ds(q * L, L)] = plsc.load_gather(
                        buf2, [parv, fq[q], colv]
                    )
            pltpu.async_copy(rowbuf, out_hbm.at[pidx], sem2).wait()

        @pl.when(c_lo < NCHUNK)
        def _():
            pltpu.async_copy(chunk_slice(c_lo), buf2.at[0], sems.at[0])

        def chunk_body(j, carry):
            c = c_lo + j
            par = j & 1

            @pl.when(c < NCHUNK)
            def _():
                pltpu.make_async_copy(
                    chunk_slice(c), buf2.at[par], sems.at[par]
                ).wait()

                @pl.when((c + 1 < NCHUNK) & (j + 1 < CPW))
                def _():
                    pltpu.async_copy(
                        chunk_slice(c + 1), buf2.at[1 - par], sems.at[1 - par]
                    )

                def rescan(g, pcnt):
                    v = lrow[pl.ds(g * L, L)]
                    p = lpos[pl.ds(g * L, L)]
                    m = (jax.lax.shift_right_logical(v, 9) == c) & (
                        g * L + lane < n_local
                    )
                    plsc.store_compressed(prow.at[pl.ds(pcnt, L)], v, mask=m)
                    plsc.store_compressed(ppos.at[pl.ds(pcnt, L)], p, mask=m)
                    pcnt = pcnt + plsc.all_reduce_population_count(m)[0]

                    def flush(pc):
                        emit_hits(par, c, L)
                        prow[pl.ds(0, L)] = prow[pl.ds(L, L)]
                        ppos[pl.ds(0, L)] = ppos[pl.ds(L, L)]
                        return pc - L

                    return lax.cond(pcnt >= L, flush, lambda pc: pc, pcnt)

                pcnt = lax.fori_loop(0, ngr, rescan, 0)

                @pl.when(pcnt > 0)
                def _():
                    emit_hits(par, c, pcnt)

            return carry

        lax.fori_loop(0, CPW, chunk_body, 0)

        @pl.when((c_lo <= NCHUNK) & (NCHUNK < c_lo + CPW))
        def _():
            pltpu.sync_copy(tail_hbm, tailbuf)

            def emit_hits_t(nvalid):
                hrv = prow[pl.ds(0, L)]
                hpv = ppos[pl.ds(0, L)]
                pidx[pl.ds(0, L)] = jnp.where(lane < nvalid, hpv, SINK)
                for r in range(L):
                    row = jnp.clip(hrv[r] - TAIL0, 0, 63)
                    for q in range(F // L):
                        rowbuf[r, pl.ds(q * L, L)] = tailbuf[row, pl.ds(q * L, L)]
                pltpu.async_copy(rowbuf, out_hbm.at[pidx], sem2).wait()

            def rescan_t(g, pcnt):
                v = lrow[pl.ds(g * L, L)]
                p = lpos[pl.ds(g * L, L)]
                m = (jax.lax.shift_right_logical(v, 9) == NCHUNK) & (
                    g * L + lane < n_local
                )
                plsc.store_compressed(prow.at[pl.ds(pcnt, L)], v, mask=m)
                plsc.store_compressed(ppos.at[pl.ds(pcnt, L)], p, mask=m)
                pcnt = pcnt + plsc.all_reduce_population_count(m)[0]

                def flush(pc):
                    emit_hits_t(L)
                    prow[pl.ds(0, L)] = prow[pl.ds(L, L)]
                    ppos[pl.ds(0, L)] = ppos[pl.ds(L, L)]
                    return pc - L

                return lax.cond(pcnt >= L, flush, lambda pc: pc, pcnt)

            pcnt = lax.fori_loop(0, ngr, rescan_t, 0)

            @pl.when(pcnt > 0)
            def _():
                emit_hits_t(pcnt)


@functools.partial(
    pl.kernel,
    out_type=jax.ShapeDtypeStruct((B,), jnp.float32),
    mesh=_mesh,
    compiler_params=pltpu.CompilerParams(
        needs_layout_passes=False, use_tc_tiling_on_sc=True
    ),
    scratch_types=[
        pltpu.VMEM((BPW // 2, 2 * F), jnp.float32),
        pltpu.VMEM((BPW // 2, 2 * F), jnp.float32),
        pltpu.VMEM((BPW,), jnp.float32),
        pltpu.VMEM((BPW,), jnp.float32),
        pltpu.VMEM((BPW,), jnp.float32),
        pltpu.VMEM((L * (L + 1),), jnp.float32),
        pltpu.SemaphoreType.DMA,
    ],
)
def _dot_kernel(urows_hbm, irows_hbm, ubg_hbm, ibg_hbm,
                out_hbm, urows, irows, ubv, ibv, outv, tbuf, sem):
    wid = lax.axis_index("s") * NC + lax.axis_index("c")
    base = wid * BPW
    HB = BPW // 2

    pltpu.sync_copy(ubg_hbm.at[pl.ds(base, BPW)], ubv)
    pltpu.sync_copy(ibg_hbm.at[pl.ds(base, BPW)], ibv)

    rowi = lax.iota(jnp.int32, L)

    for half in range(2):
        hbase = base + half * HB
        c1 = pltpu.async_copy(urows_hbm.at[pl.ds(hbase, HB), :], urows, sem)
        c2 = pltpu.async_copy(irows_hbm.at[pl.ds(hbase, HB), :], irows, sem)
        c1.wait()
        c2.wait()

        def group_body(g, carry):
            gb = g * L
            ob = half * HB + gb
            for r in range(L):
                b = gb + r
                acc = urows[b, pl.ds(0, L)] * irows[b, pl.ds(0, L)]
                for j in range(1, F // L):
                    acc = acc + urows[b, pl.ds(j * L, L)] * irows[b, pl.ds(j * L, L)]
                tbuf[pl.ds(r * (L + 1), L)] = acc
            out16 = ubv[pl.ds(ob, L)] + ibv[pl.ds(ob, L)]
            flat = rowi * (L + 1)
            for j in range(L):
                out16 = out16 + plsc.load_gather(tbuf, [flat + j])
            outv[pl.ds(ob, L)] = out16
            return carry

        lax.fori_loop(0, HB // L, group_body, 0)

    pltpu.sync_copy(outv, out_hbm.at[pl.ds(base, BPW)])


@functools.partial(
    pl.kernel,
    out_type=(
        jax.ShapeDtypeStruct((B,), jnp.float32),
        jax.ShapeDtypeStruct((B,), jnp.float32),
    ),
    mesh=_mesh,
    compiler_params=pltpu.CompilerParams(
        needs_layout_passes=False, use_tc_tiling_on_sc=False
    ),
    scratch_types=[
        pltpu.VMEM((BPW,), jnp.int32),
        pltpu.VMEM((BPW,), jnp.int32),
        pltpu.VMEM((BPW,), jnp.float32),
        pltpu.VMEM((BPW,), jnp.float32),
        pltpu.SemaphoreType.DMA,
    ],
)
def _bias_kernel(users_hbm, items_hbm, ub_hbm, ib_hbm, ubg_hbm, ibg_hbm,
                 uidx, iidx, ubv, ibv, sem):
    wid = lax.axis_index("s") * NC + lax.axis_index("c")
    base = wid * BPW
    pltpu.sync_copy(users_hbm.at[pl.ds(base, BPW)], uidx)
    pltpu.sync_copy(items_hbm.at[pl.ds(base, BPW)], iidx)
    c1 = pltpu.async_copy(ub_hbm.at[uidx], ubv, sem)
    c2 = pltpu.async_copy(ib_hbm.at[iidx], ibv, sem)
    c1.wait()
    c2.wait()
    pltpu.sync_copy(ubv, ubg_hbm.at[pl.ds(base, BPW)])
    pltpu.sync_copy(ibv, ibg_hbm.at[pl.ds(base, BPW)])


@jax.jit
def kernel(users, items, user_factors, item_factors, user_bias, item_bias):
    tfu = user_factors.T
    tfi = item_factors.T
    ub = user_bias.reshape(-1)
    ib = item_bias.reshape(-1)
    tailu = user_factors[TAIL0:, :]
    taili = item_factors[TAIL0:, :]
    urows_all, irows_all = _extract_kernel(users, items, tfu, tfi, tailu, taili)
    ubg, ibg = _bias_kernel(users, items, ub, ib)
    return _dot_kernel(urows_all, irows_all, ubg, ibg)


# speedup vs baseline: 1.0132x; 1.0095x over previous
import functools

import jax
import jax.numpy as jnp
from jax import lax
from jax.experimental import pallas as pl
from jax.experimental.pallas import tpu as pltpu
from jax.experimental.pallas import tpu_sc as plsc

L = 16
NC, NS = 2, 16
NW = NC * NS
B = 16384
F = 64
N = 1000000
BPW = B // NW
GROUPS = BPW // L

CW = 512
NCHUNK = N // CW
CPW = (NCHUNK + NW - 1) // NW
TAIL0 = NCHUNK * CW
SINK = B

_mesh = plsc.VectorSubcoreMesh(core_axis_name="c", subcore_axis_name="s")

_ROWS_T = jax.ShapeDtypeStruct((B + L, 2 * F), jnp.float32)


@functools.partial(
    pl.kernel,
    out_type=(_ROWS_T, _ROWS_T),
    mesh=_mesh,
    compiler_params=pltpu.CompilerParams(
        needs_layout_passes=False, use_tc_tiling_on_sc=True
    ),
    scratch_types=[
        pltpu.VMEM((B,), jnp.int32),
        pltpu.VMEM((B + L,), jnp.int32),
        pltpu.VMEM((B + L,), jnp.int32),
        pltpu.VMEM((3 * L,), jnp.int32),
        pltpu.VMEM((3 * L,), jnp.int32),
        pltpu.VMEM((2, F, CW), jnp.float32),
        pltpu.VMEM((2, L, 2 * F), jnp.float32),
        pltpu.VMEM((2, L), jnp.int32),
        pltpu.VMEM((F * (L + 1),), jnp.float32),
        pltpu.VMEM((64, F), jnp.float32),
        pltpu.SemaphoreType.DMA((2,)),
        pltpu.SemaphoreType.DMA((2,)),
    ],
)
def _extract_kernel(users_hbm, items_hbm, tfu_hbm, tfi_hbm, tailu_hbm,
                    taili_hbm, urows_hbm, irows_hbm,
                    allidx, lrow, lpos, prow, ppos, buf2, rowbuf, pidx, rbt,
                    tailbuf, sems, sem2s):
    wid = lax.axis_index("s") * NC + lax.axis_index("c")
    c_lo = wid * CPW

    lane = lax.iota(jnp.int32, L)
    fq = [lane + q * L for q in range(F // L)]

    for (idx_hbm, tf_hbm, tail_hbm, out_hbm) in (
        (users_hbm, tfu_hbm, tailu_hbm, urows_hbm),
        (items_hbm, tfi_hbm, taili_hbm, irows_hbm),
    ):
        pltpu.sync_copy(idx_hbm, allidx)

        def filt(g, cnt):
            v = allidx[pl.ds(g * L, L)]
            c_of = jax.lax.shift_right_logical(v, 9)
            m = (c_of >= c_lo) & (c_of < c_lo + CPW)
            plsc.store_compressed(lrow.at[pl.ds(cnt, L)], v, mask=m)
            plsc.store_compressed(lpos.at[pl.ds(cnt, L)], g * L + lane, mask=m)
            return cnt + plsc.all_reduce_population_count(m)[0]

        n_local = lax.fori_loop(0, B // L, filt, 0)
        ngr = (n_local + L - 1) // L

        def chunk_slice(c):
            return tf_hbm.at[:, pl.ds(c * CW, CW)]

        def emit_hits(par, c, nvalid, ecnt):
            slot = ecnt & 1

            @pl.when(ecnt >= 2)
            def _():
                pltpu.make_async_copy(
                    rowbuf.at[slot], out_hbm.at[pidx.at[slot]], sem2s.at[slot]
                ).wait()

            hrv = prow[pl.ds(0, L)]
            hpv = ppos[pl.ds(0, L)]
            pidx[slot, pl.ds(0, L)] = jnp.where(lane < nvalid, hpv, SINK)
            parv = jnp.full((L,), par, jnp.int32)
            colv = jnp.clip(hrv - c * CW, 0, CW - 1)
            for f in range(F):
                fv = jnp.full((L,), f, jnp.int32)
                rbt[pl.ds(f * (L + 1), L)] = plsc.load_gather(
                    buf2, [parv, fv, colv]
                )
            for r in range(L):
                for q in range(F // L):
                    rowbuf[slot, r, pl.ds(q * L, L)] = plsc.load_gather(
                        rbt, [(fq[q]) * (L + 1) + r]
                    )
            pltpu.async_copy(
                rowbuf.at[slot], out_hbm.at[pidx.at[slot]], sem2s.at[slot]
            )
            return ecnt + 1

        @pl.when(c_lo < NCHUNK)
        def _():
            pltpu.async_copy(chunk_slice(c_lo), buf2.at[0], sems.at[0])

        def chunk_body(j, ecnt0):
            c = c_lo + j
            par = j & 1

            def work(ec):
                pltpu.make_async_copy(
                    chunk_slice(c), buf2.at[par], sems.at[par]
                ).wait()

                @pl.when((c + 1 < NCHUNK) & (j + 1 < CPW))
                def _():
                    pltpu.async_copy(
                        chunk_slice(c + 1), buf2.at[1 - par], sems.at[1 - par]
                    )

                def rescan(g, carry):
                    pcnt, ec2 = carry
                    v = lrow[pl.ds(g * L, L)]
                    p = lpos[pl.ds(g * L, L)]
                    m = (jax.lax.shift_right_logical(v, 9) == c) & (
                        g * L + lane < n_local
                    )
                    plsc.store_compressed(prow.at[pl.ds(pcnt, L)], v, mask=m)
                    plsc.store_compressed(ppos.at[pl.ds(pcnt, L)], p, mask=m)
                    pcnt = pcnt + plsc.all_reduce_population_count(m)[0]

                    def flush(carry3):
                        pc, ec3 = carry3
                        ec3 = emit_hits(par, c, L, ec3)
                        prow[pl.ds(0, L)] = prow[pl.ds(L, L)]
                        ppos[pl.ds(0, L)] = ppos[pl.ds(L, L)]
                        return pc - L, ec3

                    return lax.cond(
                        pcnt >= L, flush, lambda cr: cr, (pcnt, ec2)
                    )

                pcnt, ec = lax.fori_loop(0, ngr, rescan, (0, ec))

                def tail_flush(carry4):
                    pc, ec4 = carry4
                    return pc, emit_hits(par, c, pc, ec4)

                _, ec = lax.cond(
                    pcnt > 0, tail_flush, lambda cr: cr, (pcnt, ec)
                )
                return ec

            return lax.cond(c < NCHUNK, work, lambda ec: ec, ecnt0)

        ecnt = lax.fori_loop(0, CPW, chunk_body, 0)

        for back in (2, 1):

            @pl.when(ecnt >= back)
            def _():
                slot = (ecnt - back) & 1
                pltpu.make_async_copy(
                    rowbuf.at[slot], out_hbm.at[pidx.at[slot]], sem2s.at[slot]
                ).wait()

        @pl.when((c_lo <= NCHUNK) & (NCHUNK < c_lo + CPW))
        def _():
            pltpu.sync_copy(tail_hbm, tailbuf)

            def emit_hits_t(nvalid):
                hrv = prow[pl.ds(0, L)]
                hpv = ppos[pl.ds(0, L)]
                pidx[0, pl.ds(0, L)] = jnp.where(lane < nvalid, hpv, SINK)
                for r in range(L):
                    row = jnp.clip(hrv[r] - TAIL0, 0, 63)
                    for q in range(F // L):
                        rowbuf[0, r, pl.ds(q * L, L)] = tailbuf[row, pl.ds(q * L, L)]
                pltpu.async_copy(
                    rowbuf.at[0], out_hbm.at[pidx.at[0]], sem2s.at[0]
                ).wait()

            def rescan_t(g, pcnt):
                v = lrow[pl.ds(g * L, L)]
                p = lpos[pl.ds(g * L, L)]
                m = (jax.lax.shift_right_logical(v, 9) == NCHUNK) & (
                    g * L + lane < n_local
                )
                plsc.store_compressed(prow.at[pl.ds(pcnt, L)], v, mask=m)
                plsc.store_compressed(ppos.at[pl.ds(pcnt, L)], p, mask=m)
                pcnt = pcnt + plsc.all_reduce_population_count(m)[0]

                def flush(pc):
                    emit_hits_t(L)
                    prow[pl.ds(0, L)] = prow[pl.ds(L, L)]
                    ppos[pl.ds(0, L)] = ppos[pl.ds(L, L)]
                    return pc - L

                return lax.cond(pcnt >= L, flush, lambda pc: pc, pcnt)

            pcnt = lax.fori_loop(0, ngr, rescan_t, 0)

            @pl.when(pcnt > 0)
            def _():
                emit_hits_t(pcnt)


@functools.partial(
    pl.kernel,
    out_type=jax.ShapeDtypeStruct((B,), jnp.float32),
    mesh=_mesh,
    compiler_params=pltpu.CompilerParams(
        needs_layout_passes=False, use_tc_tiling_on_sc=True
    ),
    scratch_types=[
        pltpu.VMEM((BPW // 2, 2 * F), jnp.float32),
        pltpu.VMEM((BPW // 2, 2 * F), jnp.float32),
        pltpu.VMEM((BPW,), jnp.float32),
        pltpu.VMEM((BPW,), jnp.float32),
        pltpu.VMEM((BPW,), jnp.float32),
        pltpu.VMEM((L * (L + 1),), jnp.float32),
        pltpu.SemaphoreType.DMA,
    ],
)
def _dot_kernel(urows_hbm, irows_hbm, ubg_hbm, ibg_hbm,
                out_hbm, urows, irows, ubv, ibv, outv, tbuf, sem):
    wid = lax.axis_index("s") * NC + lax.axis_index("c")
    base = wid * BPW
    HB = BPW // 2

    pltpu.sync_copy(ubg_hbm.at[pl.ds(base, BPW)], ubv)
    pltpu.sync_copy(ibg_hbm.at[pl.ds(base, BPW)], ibv)

    rowi = lax.iota(jnp.int32, L)

    for half in range(2):
        hbase = base + half * HB
        c1 = pltpu.async_copy(urows_hbm.at[pl.ds(hbase, HB), :], urows, sem)
        c2 = pltpu.async_copy(irows_hbm.at[pl.ds(hbase, HB), :], irows, sem)
        c1.wait()
        c2.wait()

        def group_body(g, carry):
            gb = g * L
            ob = half * HB + gb
            for r in range(L):
                b = gb + r
                acc = urows[b, pl.ds(0, L)] * irows[b, pl.ds(0, L)]
                for j in range(1, F // L):
                    acc = acc + urows[b, pl.ds(j * L, L)] * irows[b, pl.ds(j * L, L)]
                tbuf[pl.ds(r * (L + 1), L)] = acc
            out16 = ubv[pl.ds(ob, L)] + ibv[pl.ds(ob, L)]
            flat = rowi * (L + 1)
            for j in range(L):
                out16 = out16 + plsc.load_gather(tbuf, [flat + j])
            outv[pl.ds(ob, L)] = out16
            return carry

        lax.fori_loop(0, HB // L, group_body, 0)

    pltpu.sync_copy(outv, out_hbm.at[pl.ds(base, BPW)])


@functools.partial(
    pl.kernel,
    out_type=(
        jax.ShapeDtypeStruct((B,), jnp.float32),
        jax.ShapeDtypeStruct((B,), jnp.float32),
    ),
    mesh=_mesh,
    compiler_params=pltpu.CompilerParams(
        needs_layout_passes=False, use_tc_tiling_on_sc=False
    ),
    scratch_types=[
        pltpu.VMEM((BPW,), jnp.int32),
        pltpu.VMEM((BPW,), jnp.int32),
        pltpu.VMEM((BPW,), jnp.float32),
        pltpu.VMEM((BPW,), jnp.float32),
        pltpu.SemaphoreType.DMA,
    ],
)
def _bias_kernel(users_hbm, items_hbm, ub_hbm, ib_hbm, ubg_hbm, ibg_hbm,
                 uidx, iidx, ubv, ibv, sem):
    wid = lax.axis_index("s") * NC + lax.axis_index("c")
    base = wid * BPW
    pltpu.sync_copy(users_hbm.at[pl.ds(base, BPW)], uidx)
    pltpu.sync_copy(items_hbm.at[pl.ds(base, BPW)], iidx)
    c1 = pltpu.async_copy(ub_hbm.at[uidx], ubv, sem)
    c2 = pltpu.async_copy(ib_hbm.at[iidx], ibv, sem)
    c1.wait()
    c2.wait()
    pltpu.sync_copy(ubv, ubg_hbm.at[pl.ds(base, BPW)])
    pltpu.sync_copy(ibv, ibg_hbm.at[pl.ds(base, BPW)])


@jax.jit
def kernel(users, items, user_factors, item_factors, user_bias, item_bias):
    tfu = user_factors.T
    tfi = item_factors.T
    ub = user_bias.reshape(-1)
    ib = item_bias.reshape(-1)
    tailu = user_factors[TAIL0:, :]
    taili = item_factors[TAIL0:, :]
    urows_all, irows_all = _extract_kernel(users, items, tfu, tfi, tailu, taili)
    ubg, ibg = _bias_kernel(users, items, ub, ib)
    return _dot_kernel(urows_all, irows_all, ubg, ibg)


# speedup vs baseline: 1.0166x; 1.0034x over previous
import functools

import jax
import jax.numpy as jnp
from jax import lax
from jax.experimental import pallas as pl
from jax.experimental.pallas import tpu as pltpu
from jax.experimental.pallas import tpu_sc as plsc

L = 16
NC, NS = 2, 16
NW = NC * NS
B = 16384
F = 64
N = 1000000
BPW = B // NW
GROUPS = BPW // L

CW = 512
NCHUNK = N // CW
CPW = (NCHUNK + NW - 1) // NW
TAIL0 = NCHUNK * CW
SINK = B

_mesh = plsc.VectorSubcoreMesh(core_axis_name="c", subcore_axis_name="s")

_ROWS_T = jax.ShapeDtypeStruct((B + L, 2 * F), jnp.float32)


@functools.partial(
    pl.kernel,
    out_type=(_ROWS_T, _ROWS_T),
    mesh=_mesh,
    compiler_params=pltpu.CompilerParams(
        needs_layout_passes=False, use_tc_tiling_on_sc=True
    ),
    scratch_types=[
        pltpu.VMEM((B,), jnp.int32),
        pltpu.VMEM((B + L,), jnp.int32),
        pltpu.VMEM((B + L,), jnp.int32),
        pltpu.VMEM((3 * L,), jnp.int32),
        pltpu.VMEM((3 * L,), jnp.int32),
        pltpu.VMEM((2, F, CW), jnp.float32),
        pltpu.VMEM((2, L, 2 * F), jnp.float32),
        pltpu.VMEM((2, L), jnp.int32),
        pltpu.VMEM((F * (L + 1),), jnp.float32),
        pltpu.VMEM((64, F), jnp.float32),
        pltpu.SemaphoreType.DMA((2,)),
        pltpu.SemaphoreType.DMA((2,)),
    ],
)
def _extract_kernel(users_hbm, items_hbm, tfu_hbm, tfi_hbm, tailu_hbm,
                    taili_hbm, urows_hbm, irows_hbm,
                    allidx, lrow, lpos, prow, ppos, buf2, rowbuf, pidx, rbt,
                    tailbuf, sems, sem2s):
    wid = lax.axis_index("s") * NC + lax.axis_index("c")
    c_lo = wid * CPW

    lane = lax.iota(jnp.int32, L)
    fq = [lane + q * L for q in range(F // L)]

    for (idx_hbm, tf_hbm, tail_hbm, out_hbm) in (
        (users_hbm, tfu_hbm, tailu_hbm, urows_hbm),
        (items_hbm, tfi_hbm, taili_hbm, irows_hbm),
    ):
        pltpu.sync_copy(idx_hbm, allidx)

        def filt(g, cnt):
            v = allidx[pl.ds(g * L, L)]
            c_of = jax.lax.shift_right_logical(v, 9)
            m = (c_of >= c_lo) & (c_of < c_lo + CPW)
            plsc.store_compressed(lrow.at[pl.ds(cnt, L)], v, mask=m)
            plsc.store_compressed(lpos.at[pl.ds(cnt, L)], g * L + lane, mask=m)
            return cnt + plsc.all_reduce_population_count(m)[0]

        n_local = lax.fori_loop(0, B // L, filt, 0)
        ngr = (n_local + L - 1) // L

        def chunk_slice(c):
            return tf_hbm.at[:, pl.ds(c * CW, CW)]

        def emit_hits(par, c, nvalid, ecnt):
            slot = ecnt & 1

            @pl.when(ecnt >= 2)
            def _():
                pltpu.make_async_copy(
                    rowbuf.at[slot], out_hbm.at[pidx.at[slot]], sem2s.at[slot]
                ).wait()

            hrv = prow[pl.ds(0, L)]
            hpv = ppos[pl.ds(0, L)]
            pidx[slot, pl.ds(0, L)] = jnp.where(lane < nvalid, hpv, SINK)
            parv = jnp.full((L,), par, jnp.int32)
            colv = jnp.clip(hrv - c * CW, 0, CW - 1)

            def feat_body(f, carry):
                fv = jnp.full((L,), f, jnp.int32)
                rbt[pl.ds(f * (L + 1), L)] = plsc.load_gather(
                    buf2, [parv, fv, colv]
                )
                return carry

            lax.fori_loop(0, F, feat_body, 0)

            def hit_body(r, carry):
                for q in range(F // L):
                    rowbuf[slot, r, pl.ds(q * L, L)] = plsc.load_gather(
                        rbt, [fq[q] * (L + 1) + r]
                    )
                return carry

            lax.fori_loop(0, L, hit_body, 0)
            pltpu.async_copy(
                rowbuf.at[slot], out_hbm.at[pidx.at[slot]], sem2s.at[slot]
            )
            return ecnt + 1

        @pl.when(c_lo < NCHUNK)
        def _():
            pltpu.async_copy(chunk_slice(c_lo), buf2.at[0], sems.at[0])

        def chunk_body(j, ecnt0):
            c = c_lo + j
            par = j & 1

            def work(ec):
                pltpu.make_async_copy(
                    chunk_slice(c), buf2.at[par], sems.at[par]
                ).wait()

                @pl.when((c + 1 < NCHUNK) & (j + 1 < CPW))
                def _():
                    pltpu.async_copy(
                        chunk_slice(c + 1), buf2.at[1 - par], sems.at[1 - par]
                    )

                def rescan(g, carry):
                    pcnt, ec2 = carry
                    v = lrow[pl.ds(g * L, L)]
                    p = lpos[pl.ds(g * L, L)]
                    m = (jax.lax.shift_right_logical(v, 9) == c) & (
                        g * L + lane < n_local
                    )
                    plsc.store_compressed(prow.at[pl.ds(pcnt, L)], v, mask=m)
                    plsc.store_compressed(ppos.at[pl.ds(pcnt, L)], p, mask=m)
                    pcnt = pcnt + plsc.all_reduce_population_count(m)[0]

                    def flush(carry3):
                        pc, ec3 = carry3
                        ec3 = emit_hits(par, c, L, ec3)
                        prow[pl.ds(0, L)] = prow[pl.ds(L, L)]
                        ppos[pl.ds(0, L)] = ppos[pl.ds(L, L)]
                        return pc - L, ec3

                    return lax.cond(
                        pcnt >= L, flush, lambda cr: cr, (pcnt, ec2)
                    )

                pcnt, ec = lax.fori_loop(0, ngr, rescan, (0, ec))

                def tail_flush(carry4):
                    pc, ec4 = carry4
                    return pc, emit_hits(par, c, pc, ec4)

                _, ec = lax.cond(
                    pcnt > 0, tail_flush, lambda cr: cr, (pcnt, ec)
                )
                return ec

            return lax.cond(c < NCHUNK, work, lambda ec: ec, ecnt0)

        ecnt = lax.fori_loop(0, CPW, chunk_body, 0)

        for back in (2, 1):

            @pl.when(ecnt >= back)
            def _():
                slot = (ecnt - back) & 1
                pltpu.make_async_copy(
                    rowbuf.at[slot], out_hbm.at[pidx.at[slot]], sem2s.at[slot]
                ).wait()

        @pl.when((c_lo <= NCHUNK) & (NCHUNK < c_lo + CPW))
        def _():
            pltpu.sync_copy(tail_hbm, tailbuf)

            def emit_hits_t(nvalid):
                hrv = prow[pl.ds(0, L)]
                hpv = ppos[pl.ds(0, L)]
                pidx[0, pl.ds(0, L)] = jnp.where(lane < nvalid, hpv, SINK)
                for r in range(L):
                    row = jnp.clip(hrv[r] - TAIL0, 0, 63)
                    for q in range(F // L):
                        rowbuf[0, r, pl.ds(q * L, L)] = tailbuf[row, pl.ds(q * L, L)]
                pltpu.async_copy(
                    rowbuf.at[0], out_hbm.at[pidx.at[0]], sem2s.at[0]
                ).wait()

            def rescan_t(g, pcnt):
                v = lrow[pl.ds(g * L, L)]
                p = lpos[pl.ds(g * L, L)]
                m = (jax.lax.shift_right_logical(v, 9) == NCHUNK) & (
                    g * L + lane < n_local
                )
                plsc.store_compressed(prow.at[pl.ds(pcnt, L)], v, mask=m)
                plsc.store_compressed(ppos.at[pl.ds(pcnt, L)], p, mask=m)
                pcnt = pcnt + plsc.all_reduce_population_count(m)[0]

                def flush(pc):
                    emit_hits_t(L)
                    prow[pl.ds(0, L)] = prow[pl.ds(L, L)]
                    ppos[pl.ds(0, L)] = ppos[pl.ds(L, L)]
                    return pc - L

                return lax.cond(pcnt >= L, flush, lambda pc: pc, pcnt)

            pcnt = lax.fori_loop(0, ngr, rescan_t, 0)

            @pl.when(pcnt > 0)
            def _():
                emit_hits_t(pcnt)


@functools.partial(
    pl.kernel,
    out_type=jax.ShapeDtypeStruct((B,), jnp.float32),
    mesh=_mesh,
    compiler_params=pltpu.CompilerParams(
        needs_layout_passes=False, use_tc_tiling_on_sc=True
    ),
    scratch_types=[
        pltpu.VMEM((BPW // 2, 2 * F), jnp.float32),
        pltpu.VMEM((BPW // 2, 2 * F), jnp.float32),
        pltpu.VMEM((BPW,), jnp.float32),
        pltpu.VMEM((BPW,), jnp.float32),
        pltpu.VMEM((BPW,), jnp.float32),
        pltpu.VMEM((L * (L + 1),), jnp.float32),
        pltpu.SemaphoreType.DMA,
    ],
)
def _dot_kernel(urows_hbm, irows_hbm, ubg_hbm, ibg_hbm,
                out_hbm, urows, irows, ubv, ibv, outv, tbuf, sem):
    wid = lax.axis_index("s") * NC + lax.axis_index("c")
    base = wid * BPW
    HB = BPW // 2

    pltpu.sync_copy(ubg_hbm.at[pl.ds(base, BPW)], ubv)
    pltpu.sync_copy(ibg_hbm.at[pl.ds(base, BPW)], ibv)

    rowi = lax.iota(jnp.int32, L)

    for half in range(2):
        hbase = base + half * HB
        c1 = pltpu.async_copy(urows_hbm.at[pl.ds(hbase, HB), :], urows, sem)
        c2 = pltpu.async_copy(irows_hbm.at[pl.ds(hbase, HB), :], irows, sem)
        c1.wait()
        c2.wait()

        def group_body(g, carry):
            gb = g * L
            ob = half * HB + gb
            for r in range(L):
                b = gb + r
                acc = urows[b, pl.ds(0, L)] * irows[b, pl.ds(0, L)]
                for j in range(1, F // L):
                    acc = acc + urows[b, pl.ds(j * L, L)] * irows[b, pl.ds(j * L, L)]
                tbuf[pl.ds(r * (L + 1), L)] = acc
            out16 = ubv[pl.ds(ob, L)] + ibv[pl.ds(ob, L)]
            flat = rowi * (L + 1)
            for j in range(L):
                out16 = out16 + plsc.load_gather(tbuf, [flat + j])
            outv[pl.ds(ob, L)] = out16
            return carry

        lax.fori_loop(0, HB // L, group_body, 0)

    pltpu.sync_copy(outv, out_hbm.at[pl.ds(base, BPW)])


@functools.partial(
    pl.kernel,
    out_type=(
        jax.ShapeDtypeStruct((B,), jnp.float32),
        jax.ShapeDtypeStruct((B,), jnp.float32),
    ),
    mesh=_mesh,
    compiler_params=pltpu.CompilerParams(
        needs_layout_passes=False, use_tc_tiling_on_sc=False
    ),
    scratch_types=[
        pltpu.VMEM((BPW,), jnp.int32),
        pltpu.VMEM((BPW,), jnp.int32),
        pltpu.VMEM((BPW,), jnp.float32),
        pltpu.VMEM((BPW,), jnp.float32),
        pltpu.SemaphoreType.DMA,
    ],
)
def _bias_kernel(users_hbm, items_hbm, ub_hbm, ib_hbm, ubg_hbm, ibg_hbm,
                 uidx, iidx, ubv, ibv, sem):
    wid = lax.axis_index("s") * NC + lax.axis_index("c")
    base = wid * BPW
    pltpu.sync_copy(users_hbm.at[pl.ds(base, BPW)], uidx)
    pltpu.sync_copy(items_hbm.at[pl.ds(base, BPW)], iidx)
    c1 = pltpu.async_copy(ub_hbm.at[uidx], ubv, sem)
    c2 = pltpu.async_copy(ib_hbm.at[iidx], ibv, sem)
    c1.wait()
    c2.wait()
    pltpu.sync_copy(ubv, ubg_hbm.at[pl.ds(base, BPW)])
    pltpu.sync_copy(ibv, ibg_hbm.at[pl.ds(base, BPW)])


@jax.jit
def kernel(users, items, user_factors, item_factors, user_bias, item_bias):
    tfu = user_factors.T
    tfi = item_factors.T
    ub = user_bias.reshape(-1)
    ib = item_bias.reshape(-1)
    tailu = user_factors[TAIL0:, :]
    taili = item_factors[TAIL0:, :]
    urows_all, irows_all = _extract_kernel(users, items, tfu, tfi, tailu, taili)
    ubg, ibg = _bias_kernel(users, items, ub, ib)
    return _dot_kernel(urows_all, irows_all, ubg, ibg)


# speedup vs baseline: 1.6587x; 1.6315x over previous
import functools

import jax
import jax.numpy as jnp
from jax import lax
from jax.experimental import pallas as pl
from jax.experimental.pallas import tpu as pltpu
from jax.experimental.pallas import tpu_sc as plsc

L = 16
NC, NS = 2, 16
NW = NC * NS
B = 16384
F = 64
BPW = B // NW
GROUPS = BPW // L

_mesh = plsc.VectorSubcoreMesh(core_axis_name="c", subcore_axis_name="s")


@functools.partial(
    pl.kernel,
    out_type=jax.ShapeDtypeStruct((B,), jnp.float32),
    mesh=_mesh,
    compiler_params=pltpu.CompilerParams(
        needs_layout_passes=False, use_tc_tiling_on_sc=False
    ),
    scratch_types=[
        pltpu.VMEM((BPW,), jnp.int32),
        pltpu.VMEM((BPW,), jnp.int32),
        pltpu.VMEM((BPW, F), jnp.float32),
        pltpu.VMEM((BPW, F), jnp.float32),
        pltpu.VMEM((BPW,), jnp.float32),
        pltpu.VMEM((BPW,), jnp.float32),
        pltpu.VMEM((BPW,), jnp.float32),
        pltpu.VMEM((L * (L + 1),), jnp.float32),
        pltpu.SemaphoreType.DMA,
    ],
)
def _mf_kernel(users_hbm, items_hbm, uf_hbm, if_hbm, ub_hbm, ib_hbm, out_hbm,
               uidx, iidx, urows, irows, ubv, ibv, outv, tbuf, sem):
    wid = lax.axis_index("s") * NC + lax.axis_index("c")
    base = wid * BPW

    pltpu.sync_copy(users_hbm.at[pl.ds(base, BPW)], uidx)
    pltpu.sync_copy(items_hbm.at[pl.ds(base, BPW)], iidx)

    c1 = pltpu.async_copy(uf_hbm.at[uidx], urows, sem)
    c2 = pltpu.async_copy(if_hbm.at[iidx], irows, sem)
    c3 = pltpu.async_copy(ub_hbm.at[uidx], ubv, sem)
    c4 = pltpu.async_copy(ib_hbm.at[iidx], ibv, sem)
    c1.wait()
    c2.wait()
    c3.wait()
    c4.wait()

    rowi = lax.iota(jnp.int32, L)

    def group_body(g, carry):
        gb = g * L
        for r in range(L):
            b = gb + r
            acc = urows[b, pl.ds(0, L)] * irows[b, pl.ds(0, L)]
            for j in range(1, F // L):
                acc = acc + urows[b, pl.ds(j * L, L)] * irows[b, pl.ds(j * L, L)]
            tbuf[pl.ds(r * (L + 1), L)] = acc
        out16 = ubv[pl.ds(gb, L)] + ibv[pl.ds(gb, L)]
        flat = rowi * (L + 1)
        for j in range(L):
            out16 = out16 + plsc.load_gather(tbuf, [flat + j])
        outv[pl.ds(gb, L)] = out16
        return carry

    lax.fori_loop(0, GROUPS, group_body, 0)
    pltpu.sync_copy(outv, out_hbm.at[pl.ds(base, BPW)])


@jax.jit
def kernel(users, items, user_factors, item_factors, user_bias, item_bias):
    ub = user_bias.reshape(-1)
    ib = item_bias.reshape(-1)
    return _mf_kernel(users, items, user_factors, item_factors, ub, ib)
